# baseline (speedup 1.0000x reference)
import functools

import jax
import jax.numpy as jnp
from jax import lax
from jax.experimental import pallas as pl
from jax.experimental.pallas import tpu as pltpu
from jax.experimental.pallas import tpu_sc as plsc

_B = 8
_H = 512
_W = 512
_HW = _H * _W
_N = _B * _HW
_NW = 32
_PIX_W = _N // _NW
_T = 4096
_NCH = _PIX_W // _T
_G = _T // 16


def _build_sc_call():
    mesh = plsc.VectorSubcoreMesh(core_axis_name="c", subcore_axis_name="s")

    @functools.partial(
        pl.kernel,
        mesh=mesh,
        compiler_params=pltpu.CompilerParams(needs_layout_passes=False),
        out_type=jax.ShapeDtypeStruct((3 * _N,), jnp.float32),
        scratch_types=[
            pltpu.VMEM((_T,), jnp.float32),
            pltpu.VMEM((3 * _T,), jnp.float32),
            pltpu.VMEM((_B, 16), jnp.float32),
        ],
    )
    def unproject(depth_hbm, intr_hbm, out_hbm, d_v, o_v, k_v):
        wid = lax.axis_index("c") * 16 + lax.axis_index("s")
        b = wid // 4
        pltpu.sync_copy(intr_hbm, k_v)

        kv = k_v[b, :]
        k00 = kv[0]
        k01 = kv[1]
        k02 = kv[2]
        k10 = kv[3]
        k11 = kv[4]
        k12 = kv[5]
        k20 = kv[6]
        k21 = kv[7]
        k22 = kv[8]

        m00 = k11 * k22 - k12 * k21
        m01 = k10 * k22 - k12 * k20
        m02 = k10 * k21 - k11 * k20
        det = k00 * m00 - k01 * m01 + k02 * m02
        det_v = jnp.broadcast_to(det, (16,))
        rdet = (jnp.float32(1.0) / det_v)[0]
        a00 = m00 * rdet
        a01 = (k02 * k21 - k01 * k22) * rdet
        a02 = (k01 * k12 - k02 * k11) * rdet
        a10 = -m01 * rdet
        a11 = (k00 * k22 - k02 * k20) * rdet
        a12 = (k02 * k10 - k00 * k12) * rdet
        a20 = m02 * rdet
        a21 = (k01 * k20 - k00 * k21) * rdet
        a22 = (k00 * k11 - k01 * k10) * rdet

        iota_i = lax.iota(jnp.int32, 16)
        iota3 = iota_i * 3
        pim0 = wid * _PIX_W - b * _HW

        def chunk_body(ci, carry):
            base = wid * _PIX_W + ci * _T
            pltpu.sync_copy(depth_hbm.at[pl.ds(base, _T)], d_v)

            def group_body(g, c2):
                p = (pim0 + ci * _T + g * 16) + iota_i
                wcol = jnp.bitwise_and(p, _W - 1)
                hrow = lax.shift_right_logical(p, 9)
                xf = wcol.astype(jnp.float32) + 0.5
                yf = hrow.astype(jnp.float32) + 0.5
                cx = a00 * xf + a01 * yf + a02
                cy = a10 * xf + a11 * yf + a12
                cz = a20 * xf + a21 * yf + a22
                d = d_v[pl.ds(g * 16, 16)]
                o0 = g * 48
                plsc.store_scatter(o_v, [iota3 + o0], d * cx)
                plsc.store_scatter(o_v, [iota3 + (o0 + 1)], d * cy)
                plsc.store_scatter(o_v, [iota3 + (o0 + 2)], d * cz)
                return c2

            lax.fori_loop(0, _G, group_body, 0)
            pltpu.sync_copy(o_v, out_hbm.at[pl.ds(3 * base, 3 * _T)])
            return carry

        lax.fori_loop(0, _NCH, chunk_body, 0)

    return unproject


_unproject_call = _build_sc_call()


@jax.jit
def kernel(depth, intrinsics):
    depth_flat = depth.reshape(-1)
    intr_pad = jnp.zeros((_B, 16), jnp.float32).at[:, :9].set(
        intrinsics.reshape(_B, 9)
    )
    out_flat = _unproject_call(depth_flat, intr_pad)
    return out_flat.reshape(-1, 3)


# speedup vs baseline: 1.0044x; 1.0044x over previous
import functools

import jax
import jax.numpy as jnp
from jax import lax
from jax.experimental import pallas as pl
from jax.experimental.pallas import tpu as pltpu
from jax.experimental.pallas import tpu_sc as plsc

_B = 8
_H = 512
_W = 512
_HW = _H * _W
_N = _B * _HW
_NW = 32
_PIX_W = _N // _NW
_T = 4096
_NCH = _PIX_W // _T
_G = _T // 16


def _build_sc_call():
    mesh = plsc.VectorSubcoreMesh(core_axis_name="c", subcore_axis_name="s")

    @functools.partial(
        pl.kernel,
        mesh=mesh,
        compiler_params=pltpu.CompilerParams(needs_layout_passes=False),
        out_type=jax.ShapeDtypeStruct((3 * _N,), jnp.float32),
        scratch_types=[
            pltpu.VMEM((_T,), jnp.float32),
            pltpu.VMEM((3 * _T,), jnp.float32),
            pltpu.VMEM((_B, 16), jnp.float32),
        ],
    )
    def unproject(depth_hbm, intr_hbm, out_hbm, d_v, o_v, k_v):
        wid = lax.axis_index("c") * 16 + lax.axis_index("s")
        b = wid // 4
        pltpu.sync_copy(intr_hbm, k_v)

        kv = k_v[b, :]
        k00 = kv[0]
        k01 = kv[1]
        k02 = kv[2]
        k10 = kv[3]
        k11 = kv[4]
        k12 = kv[5]
        k20 = kv[6]
        k21 = kv[7]
        k22 = kv[8]

        m00 = k11 * k22 - k12 * k21
        m01 = k10 * k22 - k12 * k20
        m02 = k10 * k21 - k11 * k20
        det = k00 * m00 - k01 * m01 + k02 * m02
        det_v = jnp.broadcast_to(det, (16,))
        rdet = (jnp.float32(1.0) / det_v)[0]
        a00 = m00 * rdet
        a01 = (k02 * k21 - k01 * k22) * rdet
        a02 = (k01 * k12 - k02 * k11) * rdet
        a10 = -m01 * rdet
        a11 = (k00 * k22 - k02 * k20) * rdet
        a12 = (k02 * k10 - k00 * k12) * rdet
        a20 = m02 * rdet
        a21 = (k01 * k20 - k00 * k21) * rdet
        a22 = (k00 * k11 - k01 * k10) * rdet

        b0x = a02 + 0.5 * (a00 + a01)
        b0y = a12 + 0.5 * (a10 + a11)
        b0z = a22 + 0.5 * (a20 + a21)

        iota_i = lax.iota(jnp.int32, 16)
        iota3 = iota_i * 3
        iota3p1 = iota3 + 1
        iota3p2 = iota3 + 2
        pim0 = wid * _PIX_W - b * _HW

        def chunk_body(ci, carry):
            base = wid * _PIX_W + ci * _T
            pltpu.sync_copy(depth_hbm.at[pl.ds(base, _T)], d_v)

            @plsc.parallel_loop(0, _G, unroll=8)
            def group_body(g):
                p = (pim0 + ci * _T + g * 16) + iota_i
                wf = jnp.bitwise_and(p, _W - 1).astype(jnp.float32)
                hf = lax.shift_right_logical(p, 9).astype(jnp.float32)
                cx = a00 * wf + (a01 * hf + b0x)
                cy = a10 * wf + (a11 * hf + b0y)
                cz = a20 * wf + (a21 * hf + b0z)
                d = d_v[pl.ds(g * 16, 16)]
                o48 = o_v.at[pl.ds(g * 48, 48)]
                plsc.store_scatter(o48, [iota3], d * cx)
                plsc.store_scatter(o48, [iota3p1], d * cy)
                plsc.store_scatter(o48, [iota3p2], d * cz)

            pltpu.sync_copy(o_v, out_hbm.at[pl.ds(3 * base, 3 * _T)])
            return carry

        lax.fori_loop(0, _NCH, chunk_body, 0)

    return unproject


_unproject_call = _build_sc_call()


@jax.jit
def kernel(depth, intrinsics):
    depth_flat = depth.reshape(-1)
    intr_pad = jnp.zeros((_B, 16), jnp.float32).at[:, :9].set(
        intrinsics.reshape(_B, 9)
    )
    out_flat = _unproject_call(depth_flat, intr_pad)
    return out_flat.reshape(-1, 3)


# speedup vs baseline: 1.0153x; 1.0109x over previous
import functools

import jax
import jax.numpy as jnp
from jax import lax
from jax.experimental import pallas as pl
from jax.experimental.pallas import tpu as pltpu
from jax.experimental.pallas import tpu_sc as plsc

_B = 8
_H = 512
_W = 512
_HW = _H * _W
_N = _B * _HW
_NW = 32
_PIX_W = _N // _NW
_T = 4096
_NCH = _PIX_W // _T
_G = _T // 16


def _build_sc_call():
    mesh = plsc.VectorSubcoreMesh(core_axis_name="c", subcore_axis_name="s")

    @functools.partial(
        pl.kernel,
        mesh=mesh,
        compiler_params=pltpu.CompilerParams(needs_layout_passes=False),
        out_type=jax.ShapeDtypeStruct((3 * _N,), jnp.float32),
        scratch_types=[
            pltpu.VMEM((_T,), jnp.float32),
            pltpu.VMEM((3 * _T,), jnp.float32),
            pltpu.VMEM((_B, 16), jnp.float32),
        ],
    )
    def unproject(depth_hbm, intr_hbm, out_hbm, d_v, o_v, k_v):
        wid = lax.axis_index("c") * 16 + lax.axis_index("s")
        b = wid // 4
        pltpu.sync_copy(intr_hbm, k_v)

        kv = k_v[b, :]
        k00 = kv[0]
        k01 = kv[1]
        k02 = kv[2]
        k10 = kv[3]
        k11 = kv[4]
        k12 = kv[5]
        k20 = kv[6]
        k21 = kv[7]
        k22 = kv[8]

        m00 = k11 * k22 - k12 * k21
        m01 = k10 * k22 - k12 * k20
        m02 = k10 * k21 - k11 * k20
        det = k00 * m00 - k01 * m01 + k02 * m02
        det_v = jnp.broadcast_to(det, (16,))
        rdet = (jnp.float32(1.0) / det_v)[0]
        a00 = m00 * rdet
        a01 = (k02 * k21 - k01 * k22) * rdet
        a02 = (k01 * k12 - k02 * k11) * rdet
        a10 = -m01 * rdet
        a11 = (k00 * k22 - k02 * k20) * rdet
        a12 = (k02 * k10 - k00 * k12) * rdet
        a20 = m02 * rdet
        a21 = (k01 * k20 - k00 * k21) * rdet
        a22 = (k00 * k11 - k01 * k10) * rdet

        b0x = a02 + 0.5 * (a00 + a01)
        b0y = a12 + 0.5 * (a10 + a11)
        b0z = a22 + 0.5 * (a20 + a21)

        iota_i = lax.iota(jnp.int32, 16)
        iota3 = iota_i * 3
        iota3p1 = iota3 + 1
        iota3p2 = iota3 + 2
        pim0 = wid * _PIX_W - b * _HW

        def chunk_body(ci, carry):
            base = wid * _PIX_W + ci * _T
            pltpu.sync_copy(depth_hbm.at[pl.ds(base, _T)], d_v)

            @plsc.parallel_loop(0, 1, unroll=1)
            def group_body(g):
                p = (pim0 + ci * _T + g * 16) + iota_i
                wf = jnp.bitwise_and(p, _W - 1).astype(jnp.float32)
                hf = lax.shift_right_logical(p, 9).astype(jnp.float32)
                cx = a00 * wf + (a01 * hf + b0x)
                cy = a10 * wf + (a11 * hf + b0y)
                cz = a20 * wf + (a21 * hf + b0z)
                d = d_v[pl.ds(g * 16, 16)]
                o_v[pl.ds(g * 48, 16)] = d * cx
                o_v[pl.ds(g * 48 + 16, 16)] = d * cy
                o_v[pl.ds(g * 48 + 32, 16)] = d * cz

            pltpu.sync_copy(o_v, out_hbm.at[pl.ds(3 * base, 3 * _T)])
            return carry

        lax.fori_loop(0, _NCH, chunk_body, 0)

    return unproject


_unproject_call = _build_sc_call()


@jax.jit
def kernel(depth, intrinsics):
    depth_flat = depth.reshape(-1)
    intr_pad = jnp.zeros((_B, 16), jnp.float32).at[:, :9].set(
        intrinsics.reshape(_B, 9)
    )
    out_flat = _unproject_call(depth_flat, intr_pad)
    return out_flat.reshape(-1, 3)


# speedup vs baseline: 19.7228x; 19.4256x over previous
import functools

import jax
import jax.numpy as jnp
from jax import lax
from jax.experimental import pallas as pl
from jax.experimental.pallas import tpu as pltpu
from jax.experimental.pallas import tpu_sc as plsc

_B = 8
_H = 512
_W = 512
_HW = _H * _W
_N = _B * _HW
_NW = 32
_PIX_W = _N // _NW
_T = 4096
_NCH = _PIX_W // _T
_G = _T // 16


def _build_sc_call():
    mesh = plsc.VectorSubcoreMesh(core_axis_name="c", subcore_axis_name="s")

    @functools.partial(
        pl.kernel,
        mesh=mesh,
        compiler_params=pltpu.CompilerParams(needs_layout_passes=False),
        out_type=jax.ShapeDtypeStruct((4 * _N,), jnp.float32),
        scratch_types=[
            pltpu.VMEM((_T,), jnp.float32),
            pltpu.VMEM((4 * _T,), jnp.float32),
            pltpu.VMEM((_B, 16), jnp.float32),
        ],
    )
    def unproject(depth_hbm, intr_hbm, out_hbm, d_v, o_v, k_v):
        wid = lax.axis_index("c") * 16 + lax.axis_index("s")
        b = wid // 4
        pltpu.sync_copy(intr_hbm, k_v)

        kv = k_v[b, :]
        k00 = kv[0]
        k01 = kv[1]
        k02 = kv[2]
        k10 = kv[3]
        k11 = kv[4]
        k12 = kv[5]
        k20 = kv[6]
        k21 = kv[7]
        k22 = kv[8]

        m00 = k11 * k22 - k12 * k21
        m01 = k10 * k22 - k12 * k20
        m02 = k10 * k21 - k11 * k20
        det = k00 * m00 - k01 * m01 + k02 * m02
        det_v = jnp.broadcast_to(det, (16,))
        rdet = (jnp.float32(1.0) / det_v)[0]
        a00 = m00 * rdet
        a01 = (k02 * k21 - k01 * k22) * rdet
        a02 = (k01 * k12 - k02 * k11) * rdet
        a10 = -m01 * rdet
        a11 = (k00 * k22 - k02 * k20) * rdet
        a12 = (k02 * k10 - k00 * k12) * rdet
        a20 = m02 * rdet
        a21 = (k01 * k20 - k00 * k21) * rdet
        a22 = (k00 * k11 - k01 * k10) * rdet

        b0x = a02 + 0.5 * (a00 + a01)
        b0y = a12 + 0.5 * (a10 + a11)
        b0z = a22 + 0.5 * (a20 + a21)

        iota_i = lax.iota(jnp.int32, 16)
        pim0 = wid * _PIX_W - b * _HW

        def chunk_body(ci, carry):
            base = wid * _PIX_W + ci * _T
            pltpu.sync_copy(depth_hbm.at[pl.ds(base, _T)], d_v)

            @plsc.parallel_loop(0, _G, unroll=8)
            def group_body(g):
                p = (pim0 + ci * _T + g * 16) + iota_i
                wf = jnp.bitwise_and(p, _W - 1).astype(jnp.float32)
                hf = lax.shift_right_logical(p, 9).astype(jnp.float32)
                cx = a00 * wf + (a01 * hf + b0x)
                cy = a10 * wf + (a11 * hf + b0y)
                cz = a20 * wf + (a21 * hf + b0z)
                d = d_v[pl.ds(g * 16, 16)]
                off = (lax.shift_right_logical(g, 3) * 512
                       + jnp.bitwise_and(g, 7) * 16)
                o_v[pl.ds(off, 16)] = d * cx
                o_v[pl.ds(off + 128, 16)] = d * cy
                o_v[pl.ds(off + 256, 16)] = d * cz

            pltpu.sync_copy(o_v, out_hbm.at[pl.ds(4 * base, 4 * _T)])
            return carry

        lax.fori_loop(0, _NCH, chunk_body, 0)

    return unproject


_unproject_call = _build_sc_call()


@jax.jit
def kernel(depth, intrinsics):
    depth_flat = depth.reshape(-1)
    intr_pad = jnp.zeros((_B, 16), jnp.float32).at[:, :9].set(
        intrinsics.reshape(_B, 9)
    )
    out_rec = _unproject_call(depth_flat, intr_pad)
    out4 = (
        out_rec.reshape(_N // 128, 4, 128)
        .transpose(0, 2, 1)
        .reshape(_N, 4)
    )
    return out4[:, :3]


# speedup vs baseline: 27.1589x; 1.3770x over previous
import functools

import jax
import jax.numpy as jnp
from jax import lax
from jax.experimental import pallas as pl
from jax.experimental.pallas import tpu as pltpu
from jax.experimental.pallas import tpu_sc as plsc

_B = 8
_H = 512
_W = 512
_HW = _H * _W
_N = _B * _HW
_NW = 32
_PIX_W = _N // _NW
_T = 8192
_NCH = _PIX_W // _T
_G = _T // 16


def _build_sc_call():
    mesh = plsc.VectorSubcoreMesh(core_axis_name="c", subcore_axis_name="s")

    @functools.partial(
        pl.kernel,
        mesh=mesh,
        compiler_params=pltpu.CompilerParams(needs_layout_passes=False),
        out_type=jax.ShapeDtypeStruct((4 * _N,), jnp.float32),
        scratch_types=[
            pltpu.VMEM((2 * _T,), jnp.float32),
            pltpu.VMEM((8 * _T,), jnp.float32),
            pltpu.VMEM((_B, 16), jnp.float32),
            pltpu.SemaphoreType.DMA,
            pltpu.SemaphoreType.DMA,
            pltpu.SemaphoreType.DMA,
            pltpu.SemaphoreType.DMA,
        ],
    )
    def unproject(depth_hbm, intr_hbm, out_hbm, d_v, o_v, k_v,
                  sin0, sin1, sout0, sout1):
        wid = lax.axis_index("c") * 16 + lax.axis_index("s")
        b = wid // 4
        pltpu.sync_copy(intr_hbm, k_v)

        kv = k_v[b, :]
        k00 = kv[0]
        k01 = kv[1]
        k02 = kv[2]
        k10 = kv[3]
        k11 = kv[4]
        k12 = kv[5]
        k20 = kv[6]
        k21 = kv[7]
        k22 = kv[8]

        m00 = k11 * k22 - k12 * k21
        m01 = k10 * k22 - k12 * k20
        m02 = k10 * k21 - k11 * k20
        det = k00 * m00 - k01 * m01 + k02 * m02
        det_v = jnp.broadcast_to(det, (16,))
        rdet = (jnp.float32(1.0) / det_v)[0]
        a00 = m00 * rdet
        a01 = (k02 * k21 - k01 * k22) * rdet
        a02 = (k01 * k12 - k02 * k11) * rdet
        a10 = -m01 * rdet
        a11 = (k00 * k22 - k02 * k20) * rdet
        a12 = (k02 * k10 - k00 * k12) * rdet
        a20 = m02 * rdet
        a21 = (k01 * k20 - k00 * k21) * rdet
        a22 = (k00 * k11 - k01 * k10) * rdet

        b0x = a02 + 0.5 * (a00 + a01)
        b0y = a12 + 0.5 * (a10 + a11)
        b0z = a22 + 0.5 * (a20 + a21)

        iota_i = lax.iota(jnp.int32, 16)
        pim0 = wid * _PIX_W - b * _HW
        sins = (sin0, sin1)
        souts = (sout0, sout1)

        def start_in(ci):
            base = wid * _PIX_W + ci * _T
            return pltpu.async_copy(
                depth_hbm.at[pl.ds(base, _T)],
                d_v.at[pl.ds((ci % 2) * _T, _T)],
                sins[ci % 2],
            )

        def start_out(ci):
            base = wid * _PIX_W + ci * _T
            return pltpu.async_copy(
                o_v.at[pl.ds((ci % 2) * 4 * _T, 4 * _T)],
                out_hbm.at[pl.ds(4 * base, 4 * _T)],
                souts[ci % 2],
            )

        in_cp = [start_in(0)]
        out_cp = [None, None]
        for ci in range(_NCH):
            slot = ci % 2
            if ci + 1 < _NCH:
                in_cp.append(start_in(ci + 1))
            in_cp[ci].wait()
            if out_cp[slot] is not None:
                out_cp[slot].wait()
            d_off = slot * _T
            o_off = slot * 4 * _T

            @plsc.parallel_loop(0, _G, unroll=8)
            def group_body(g):
                p = (pim0 + ci * _T + g * 16) + iota_i
                wf = jnp.bitwise_and(p, _W - 1).astype(jnp.float32)
                hf = lax.shift_right_logical(p, 9).astype(jnp.float32)
                cx = a00 * wf + (a01 * hf + b0x)
                cy = a10 * wf + (a11 * hf + b0y)
                cz = a20 * wf + (a21 * hf + b0z)
                d = d_v[pl.ds(d_off + g * 16, 16)]
                off = (o_off
                       + lax.shift_right_logical(g, 3) * 512
                       + jnp.bitwise_and(g, 7) * 16)
                o_v[pl.ds(off, 16)] = d * cx
                o_v[pl.ds(off + 128, 16)] = d * cy
                o_v[pl.ds(off + 256, 16)] = d * cz

            out_cp[slot] = start_out(ci)
        out_cp[(_NCH - 1) % 2].wait()
        out_cp[_NCH % 2].wait()

    return unproject


_unproject_call = _build_sc_call()


@jax.jit
def kernel(depth, intrinsics):
    depth_flat = depth.reshape(-1)
    intr_pad = jnp.zeros((_B, 16), jnp.float32).at[:, :9].set(
        intrinsics.reshape(_B, 9)
    )
    out_rec = _unproject_call(depth_flat, intr_pad)
    out4 = (
        out_rec.reshape(_N // 128, 4, 128)
        .transpose(0, 2, 1)
        .reshape(_N, 4)
    )
    return out4[:, :3]


# speedup vs baseline: 37.4631x; 1.3794x over previous
import functools

import jax
import jax.numpy as jnp
from jax import lax
from jax.experimental import pallas as pl
from jax.experimental.pallas import tpu as pltpu
from jax.experimental.pallas import tpu_sc as plsc

_B = 8
_H = 512
_W = 512
_HW = _H * _W
_N = _B * _HW
_NW = 32
_PIX_W = _N // _NW
_T = 8192
_NCH = _PIX_W // _T
_G = _T // 16


def _build_sc_call():
    mesh = plsc.VectorSubcoreMesh(core_axis_name="c", subcore_axis_name="s")

    @functools.partial(
        pl.kernel,
        mesh=mesh,
        compiler_params=pltpu.CompilerParams(needs_layout_passes=False),
        out_type=jax.ShapeDtypeStruct((4 * _N,), jnp.float32),
        scratch_types=[
            pltpu.VMEM((2 * _T,), jnp.float32),
            pltpu.VMEM((8 * _T,), jnp.float32),
            pltpu.VMEM((_B, 16), jnp.float32),
            pltpu.SemaphoreType.DMA,
            pltpu.SemaphoreType.DMA,
            pltpu.SemaphoreType.DMA,
            pltpu.SemaphoreType.DMA,
        ],
    )
    def unproject(depth_hbm, intr_hbm, out_hbm, d_v, o_v, k_v,
                  sin0, sin1, sout0, sout1):
        wid = lax.axis_index("c") * 16 + lax.axis_index("s")
        b = wid // 4
        pltpu.sync_copy(intr_hbm, k_v)

        kv = k_v[b, :]
        k00 = kv[0]
        k01 = kv[1]
        k02 = kv[2]
        k10 = kv[3]
        k11 = kv[4]
        k12 = kv[5]
        k20 = kv[6]
        k21 = kv[7]
        k22 = kv[8]

        m00 = k11 * k22 - k12 * k21
        m01 = k10 * k22 - k12 * k20
        m02 = k10 * k21 - k11 * k20
        det = k00 * m00 - k01 * m01 + k02 * m02
        det_v = jnp.broadcast_to(det, (16,))
        rdet = (jnp.float32(1.0) / det_v)[0]
        a00 = m00 * rdet
        a01 = (k02 * k21 - k01 * k22) * rdet
        a02 = (k01 * k12 - k02 * k11) * rdet
        a10 = -m01 * rdet
        a11 = (k00 * k22 - k02 * k20) * rdet
        a12 = (k02 * k10 - k00 * k12) * rdet
        a20 = m02 * rdet
        a21 = (k01 * k20 - k00 * k21) * rdet
        a22 = (k00 * k11 - k01 * k10) * rdet

        b0x = a02 + 0.5 * (a00 + a01)
        b0y = a12 + 0.5 * (a10 + a11)
        b0z = a22 + 0.5 * (a20 + a21)

        iota_i = lax.iota(jnp.int32, 16)
        pim0 = wid * _PIX_W - b * _HW
        sins = (sin0, sin1)
        souts = (sout0, sout1)

        def start_in(ci):
            base = wid * _PIX_W + ci * _T
            return pltpu.async_copy(
                depth_hbm.at[pl.ds(base, _T)],
                d_v.at[pl.ds((ci % 2) * _T, _T)],
                sins[ci % 2],
            )

        def start_out(ci):
            base = wid * _PIX_W + ci * _T
            return pltpu.async_copy(
                o_v.at[pl.ds((ci % 2) * 4 * _T, 4 * _T)],
                out_hbm.at[pl.ds(4 * base, 4 * _T)],
                souts[ci % 2],
            )

        in_cp = [start_in(0)]
        out_cp = [None, None]
        for ci in range(_NCH):
            slot = ci % 2
            if ci + 1 < _NCH:
                in_cp.append(start_in(ci + 1))
            in_cp[ci].wait()
            if out_cp[slot] is not None:
                out_cp[slot].wait()
            d_off = slot * _T
            o_off = slot * 4 * _T

            @plsc.parallel_loop(0, _G, unroll=8)
            def group_body(g):
                s = pim0 + ci * _T + g * 16
                w0 = (jnp.bitwise_and(lax.shift_right_logical(s, 10), 3) * 128
                      + jnp.bitwise_and(s, 127))
                h = (jnp.bitwise_and(lax.shift_right_logical(s, 12), 63) * 8
                     + jnp.bitwise_and(lax.shift_right_logical(s, 7), 7))
                wf = (w0 + iota_i).astype(jnp.float32)
                hf = lax.broadcast(h, (16,)).astype(jnp.float32)
                cx = a00 * wf + (a01 * hf + b0x)
                cy = a10 * wf + (a11 * hf + b0y)
                cz = a20 * wf + (a21 * hf + b0z)
                d = d_v[pl.ds(d_off + g * 16, 16)]
                off = (o_off
                       + jnp.bitwise_and(lax.shift_right_logical(g, 8), 1)
                       * 16384
                       + jnp.bitwise_and(lax.shift_right_logical(g, 3), 7)
                       * 2048
                       + jnp.bitwise_and(lax.shift_right_logical(g, 6), 3)
                       * 512
                       + jnp.bitwise_and(g, 7) * 16)
                o_v[pl.ds(off, 16)] = d * cx
                o_v[pl.ds(off + 128, 16)] = d * cy
                o_v[pl.ds(off + 256, 16)] = d * cz

            out_cp[slot] = start_out(ci)
        out_cp[(_NCH - 1) % 2].wait()
        out_cp[_NCH % 2].wait()

    return unproject


_unproject_call = _build_sc_call()


@jax.jit
def kernel(depth, intrinsics):
    depth_flat = (
        depth.reshape(_B, _H // 8, 8, _W // 128, 128)
        .transpose(0, 1, 3, 2, 4)
        .reshape(-1)
    )
    intr_pad = jnp.zeros((_B, 16), jnp.float32).at[:, :9].set(
        intrinsics.reshape(_B, 9)
    )
    out_rec = _unproject_call(depth_flat, intr_pad)
    out4 = (
        out_rec.reshape(_N // 128, 4, 128)
        .transpose(0, 2, 1)
        .reshape(_N, 4)
    )
    return out4[:, :3]


# speedup vs baseline: 38.8396x; 1.0367x over previous
import functools

import jax
import jax.numpy as jnp
from jax import lax
from jax.experimental import pallas as pl
from jax.experimental.pallas import tpu as pltpu
from jax.experimental.pallas import tpu_sc as plsc

_B = 8
_H = 512
_W = 512
_HW = _H * _W
_N = _B * _HW
_NW = 32
_PIX_W = _N // _NW
_T = 8192
_NCH = _PIX_W // _T
_G = _T // 16


def _build_sc_call():
    mesh = plsc.VectorSubcoreMesh(core_axis_name="c", subcore_axis_name="s")

    @functools.partial(
        pl.kernel,
        mesh=mesh,
        compiler_params=pltpu.CompilerParams(needs_layout_passes=False),
        out_type=jax.ShapeDtypeStruct((_N // 128, 4, 128), jnp.float32),
        scratch_types=[
            pltpu.VMEM((2 * _T,), jnp.float32),
            pltpu.VMEM((2 * 3 * (_T // 128), 1, 128), jnp.float32),
            pltpu.VMEM((_B, 16), jnp.float32),
            pltpu.SemaphoreType.DMA,
            pltpu.SemaphoreType.DMA,
            pltpu.SemaphoreType.DMA,
            pltpu.SemaphoreType.DMA,
        ],
    )
    def unproject(depth_hbm, intr_hbm, out_hbm, d_v, o_v, k_v,
                  sin0, sin1, sout0, sout1):
        wid = lax.axis_index("c") * 16 + lax.axis_index("s")
        b = wid // 4
        pltpu.sync_copy(intr_hbm, k_v)

        kv = k_v[b, :]
        k00 = kv[0]
        k01 = kv[1]
        k02 = kv[2]
        k10 = kv[3]
        k11 = kv[4]
        k12 = kv[5]
        k20 = kv[6]
        k21 = kv[7]
        k22 = kv[8]

        m00 = k11 * k22 - k12 * k21
        m01 = k10 * k22 - k12 * k20
        m02 = k10 * k21 - k11 * k20
        det = k00 * m00 - k01 * m01 + k02 * m02
        det_v = jnp.broadcast_to(det, (16,))
        rdet = (jnp.float32(1.0) / det_v)[0]
        a00 = m00 * rdet
        a01 = (k02 * k21 - k01 * k22) * rdet
        a02 = (k01 * k12 - k02 * k11) * rdet
        a10 = -m01 * rdet
        a11 = (k00 * k22 - k02 * k20) * rdet
        a12 = (k02 * k10 - k00 * k12) * rdet
        a20 = m02 * rdet
        a21 = (k01 * k20 - k00 * k21) * rdet
        a22 = (k00 * k11 - k01 * k10) * rdet

        b0x = a02 + 0.5 * (a00 + a01)
        b0y = a12 + 0.5 * (a10 + a11)
        b0z = a22 + 0.5 * (a20 + a21)

        iota_i = lax.iota(jnp.int32, 16)
        pim0 = wid * _PIX_W - b * _HW
        sins = (sin0, sin1)
        souts = (sout0, sout1)

        def start_in(ci):
            base = wid * _PIX_W + ci * _T
            return pltpu.async_copy(
                depth_hbm.at[pl.ds(base, _T)],
                d_v.at[pl.ds((ci % 2) * _T, _T)],
                sins[ci % 2],
            )

        nblk = _T // 128

        def start_out(ci):
            blk0 = (wid * _PIX_W + ci * _T) // 128
            slot = ci % 2
            return [
                pltpu.async_copy(
                    o_v.at[pl.ds((slot * 3 + p) * nblk, nblk), :, :],
                    out_hbm.at[pl.ds(blk0, nblk), pl.ds(p, 1), :],
                    souts[slot],
                )
                for p in range(3)
            ]

        in_cp = [start_in(0)]
        out_cp = [None, None]
        for ci in range(_NCH):
            slot = ci % 2
            if ci + 1 < _NCH:
                in_cp.append(start_in(ci + 1))
            in_cp[ci].wait()
            if out_cp[slot] is not None:
                for cp in out_cp[slot]:
                    cp.wait()
            d_off = slot * _T

            @plsc.parallel_loop(0, _G, unroll=4)
            def group_body(g):
                s = pim0 + ci * _T + g * 16
                w0 = (jnp.bitwise_and(lax.shift_right_logical(s, 10), 3) * 128
                      + jnp.bitwise_and(s, 127))
                h = (jnp.bitwise_and(lax.shift_right_logical(s, 12), 63) * 8
                     + jnp.bitwise_and(lax.shift_right_logical(s, 7), 7))
                wf = (w0 + iota_i).astype(jnp.float32)
                hf = lax.broadcast(h, (16,)).astype(jnp.float32)
                cx = a00 * wf + (a01 * hf + b0x)
                cy = a10 * wf + (a11 * hf + b0y)
                cz = a20 * wf + (a21 * hf + b0z)
                d = d_v[pl.ds(d_off + g * 16, 16)]
                blk = (jnp.bitwise_and(lax.shift_right_logical(g, 8), 1) * 32
                       + jnp.bitwise_and(lax.shift_right_logical(g, 3), 7) * 4
                       + jnp.bitwise_and(lax.shift_right_logical(g, 6), 3))
                l0 = jnp.bitwise_and(g, 7) * 16
                row = slot * (3 * _G // 8) + blk
                o_v[row, 0, pl.ds(l0, 16)] = d * cx
                o_v[row + _G // 8, 0, pl.ds(l0, 16)] = d * cy
                o_v[row + _G // 4, 0, pl.ds(l0, 16)] = d * cz

            out_cp[slot] = start_out(ci)
        for cps in out_cp:
            if cps is not None:
                for cp in cps:
                    cp.wait()

    return unproject


_unproject_call = _build_sc_call()


@jax.jit
def kernel(depth, intrinsics):
    depth_flat = (
        depth.reshape(_B, _H // 8, 8, _W // 128, 128)
        .transpose(0, 1, 3, 2, 4)
        .reshape(-1)
    )
    intr_pad = jnp.pad(intrinsics.reshape(_B, 9), ((0, 0), (0, 7)))
    out_rec = _unproject_call(depth_flat, intr_pad)
    out4 = out_rec.transpose(0, 2, 1).reshape(_N, 4)
    return out4[:, :3]
